# baseline (speedup 1.0000x reference)
import functools

import jax
import jax.numpy as jnp
from jax import lax
from jax.experimental import pallas as pl
from jax.experimental.pallas import tpu as pltpu
from jax.experimental.pallas import tpu_sc as plsc

N = 8192
K = 8192
D = 256

BN = 512
BK = 512
NJ = K // BK

SN = 32
SL = 128

H = 1

_NC = 2
_NS = 16
_NW = _NC * _NS

_sc_mesh = plsc.VectorSubcoreMesh(core_axis_name="c", subcore_axis_name="s")


def _argmax_body(x_ref, at_ref, out_ref, vmax_ref, vidx_ref,
                 ab0_ref, ab1_ref, ab2_ref, xb0_ref, xb1_ref, xb2_ref):
    i = pl.program_id(0)
    j2 = pl.program_id(1)
    jA = 2 * j2
    jB = 2 * j2 + 1

    @pl.when((i == 0) & (j2 == 0))
    def _build_ab():
        ab0_ref[...] = jnp.broadcast_to(at_ref[0:1, :], (SN, K))
        ab1_ref[...] = jnp.broadcast_to(at_ref[1:2, :], (SN, K))
        ab2_ref[...] = jnp.broadcast_to(at_ref[2:3, :], (SN, K))

    @pl.when(j2 == 0)
    def _init():
        vmax_ref[...] = jnp.full((BN, BK), -jnp.inf, jnp.float32)
        vidx_ref[...] = jnp.zeros((BN, BK), jnp.int32)
        xb0_ref[...] = jnp.broadcast_to(x_ref[:, 0:1], (BN, SL))
        xb1_ref[...] = jnp.broadcast_to(x_ref[:, 1:2], (BN, SL))
        xb2_ref[...] = jnp.broadcast_to(x_ref[:, 2:3], (BN, SL))

    for li in range(BK // SL):
        cols = pl.ds(li * SL, SL)
        aAcols = pl.ds(jA * BK + li * SL, SL)
        aBcols = pl.ds(jB * BK + li * SL, SL)
        aA0 = ab0_ref[:, aAcols]
        aA1 = ab1_ref[:, aAcols]
        aA2 = ab2_ref[:, aAcols]
        aB0 = ab0_ref[:, aBcols]
        aB1 = ab1_ref[:, aBcols]
        aB2 = ab2_ref[:, aBcols]
        for ni in range(BN // SN):
            rows = pl.ds(ni * SN, SN)
            x0 = xb0_ref[rows, :]
            x1 = xb1_ref[rows, :]
            x2 = xb2_ref[rows, :]
            dA0 = x0 - aA0
            dA1 = x1 - aA1
            dA2 = x2 - aA2
            distA = dA0 * dA0 + dA1 * dA1 + dA2 * dA2
            dB0 = x0 - aB0
            dB1 = x1 - aB1
            dB2 = x2 - aB2
            distB = dB0 * dB0 + dB1 * dB1 + dB2 * dB2
            updB = distB > distA
            distP = jnp.maximum(distA, distB)
            jP = jnp.where(updB, jB, jA)
            cur = vmax_ref[rows, cols]
            upd = distP > cur
            vmax_ref[rows, cols] = jnp.maximum(distP, cur)
            vidx_ref[rows, cols] = jnp.where(upd, jP, vidx_ref[rows, cols])

    @pl.when(j2 == NJ // 2 - 1)
    def _finish():
        vm = vmax_ref[...]
        lane = lax.broadcasted_iota(jnp.int32, (BN, BK), 1)
        vi = vidx_ref[...] * BK + lane
        m = jnp.max(vm, axis=1, keepdims=True)
        idx = jnp.min(jnp.where(vm == m, vi, 2**31 - 1), axis=1)
        out_ref[...] = idx.reshape(1, 1, BN)


def _make_argmax_call(rows):
    ni = rows // BN
    return pl.pallas_call(
        _argmax_body,
        grid=(ni, NJ // 2),
        in_specs=[
            pl.BlockSpec((BN, 3), lambda i, j: (i, 0)),
            pl.BlockSpec((3, K), lambda i, j: (0, 0)),
        ],
        out_specs=pl.BlockSpec((1, 1, BN), lambda i, j: (i, 0, 0)),
        out_shape=jax.ShapeDtypeStruct((ni, 1, BN), jnp.int32),
        scratch_shapes=[
            pltpu.VMEM((BN, BK), jnp.float32),
            pltpu.VMEM((BN, BK), jnp.int32),
            pltpu.VMEM((SN, K), jnp.float32),
            pltpu.VMEM((SN, K), jnp.float32),
            pltpu.VMEM((SN, K), jnp.float32),
            pltpu.VMEM((BN, SL), jnp.float32),
            pltpu.VMEM((BN, SL), jnp.float32),
            pltpu.VMEM((BN, SL), jnp.float32),
        ],
        compiler_params=pltpu.CompilerParams(
            dimension_semantics=("parallel", "arbitrary"),
        ),
    )


def _make_sc_gather(rows):
    bpw = rows // _NW
    nch = 4
    ch = bpw // nch

    @functools.partial(
        pl.kernel,
        mesh=_sc_mesh,
        out_type=jax.ShapeDtypeStruct((rows, D), jnp.float32),
        scratch_types=[
            pltpu.VMEM((bpw,), jnp.int32),
            pltpu.VMEM((bpw, D), jnp.float32),
            [pltpu.SemaphoreType.DMA] * nch,
            pltpu.SemaphoreType.DMA,
        ],
    )
    def gather(table_hbm, idx_hbm, out_hbm, idx_v, rows_v, gsems, ssem):
        wid = lax.axis_index("s") * _NC + lax.axis_index("c")
        base = wid * bpw
        pltpu.sync_copy(idx_hbm.at[pl.ds(base, bpw)], idx_v)
        gathers = []
        for c in range(nch):
            gathers.append(
                pltpu.async_copy(
                    table_hbm.at[idx_v.at[pl.ds(c * ch, ch)]],
                    rows_v.at[pl.ds(c * ch, ch)],
                    gsems[c],
                )
            )
        scatters = []
        for c in range(nch):
            gathers[c].wait()
            scatters.append(
                pltpu.async_copy(
                    rows_v.at[pl.ds(c * ch, ch)],
                    out_hbm.at[pl.ds(base + c * ch, ch)],
                    ssem,
                )
            )
        for s in scatters:
            s.wait()

    return gather


_NH = N // H
_argmax_call = _make_argmax_call(_NH)
_sc_gather = _make_sc_gather(_NH)


def kernel(x, anchors, codebook):
    at = anchors.T
    outs = []
    for h in range(H):
        xs = lax.slice(x, (h * _NH, 0), ((h + 1) * _NH, 3))
        idx = _argmax_call(xs, at).reshape(_NH)
        outs.append(_sc_gather(codebook, idx))
    return jnp.concatenate(outs, axis=0)


# speedup vs baseline: 1.0745x; 1.0745x over previous
import functools

import jax
import jax.numpy as jnp
from jax import lax
from jax.experimental import pallas as pl
from jax.experimental.pallas import tpu as pltpu
from jax.experimental.pallas import tpu_sc as plsc

N = 8192
K = 8192
D = 256

BN = 512
BK = 512
NJ = K // BK

SN = 32
SL = 128

H = 2

_NC = 2
_NS = 16
_NW = _NC * _NS

_sc_mesh = plsc.VectorSubcoreMesh(core_axis_name="c", subcore_axis_name="s")


def _argmax_body(x_ref, at_ref, out_ref, vmax_ref, vidx_ref,
                 ab0_ref, ab1_ref, ab2_ref, xb0_ref, xb1_ref, xb2_ref):
    i = pl.program_id(0)
    j2 = pl.program_id(1)
    jA = 2 * j2
    jB = 2 * j2 + 1

    @pl.when((i == 0) & (j2 == 0))
    def _build_ab():
        ab0_ref[...] = jnp.broadcast_to(at_ref[0:1, :], (SN, K))
        ab1_ref[...] = jnp.broadcast_to(at_ref[1:2, :], (SN, K))
        ab2_ref[...] = jnp.broadcast_to(at_ref[2:3, :], (SN, K))

    @pl.when(j2 == 0)
    def _init():
        vmax_ref[...] = jnp.full((BN, BK), -jnp.inf, jnp.float32)
        vidx_ref[...] = jnp.zeros((BN, BK), jnp.int32)
        xb0_ref[...] = jnp.broadcast_to(x_ref[:, 0:1], (BN, SL))
        xb1_ref[...] = jnp.broadcast_to(x_ref[:, 1:2], (BN, SL))
        xb2_ref[...] = jnp.broadcast_to(x_ref[:, 2:3], (BN, SL))

    for li in range(BK // SL):
        cols = pl.ds(li * SL, SL)
        aAcols = pl.ds(jA * BK + li * SL, SL)
        aBcols = pl.ds(jB * BK + li * SL, SL)
        aA0 = ab0_ref[:, aAcols]
        aA1 = ab1_ref[:, aAcols]
        aA2 = ab2_ref[:, aAcols]
        aB0 = ab0_ref[:, aBcols]
        aB1 = ab1_ref[:, aBcols]
        aB2 = ab2_ref[:, aBcols]
        for ni in range(BN // SN):
            rows = pl.ds(ni * SN, SN)
            x0 = xb0_ref[rows, :]
            x1 = xb1_ref[rows, :]
            x2 = xb2_ref[rows, :]
            dA0 = x0 - aA0
            dA1 = x1 - aA1
            dA2 = x2 - aA2
            distA = dA0 * dA0 + dA1 * dA1 + dA2 * dA2
            dB0 = x0 - aB0
            dB1 = x1 - aB1
            dB2 = x2 - aB2
            distB = dB0 * dB0 + dB1 * dB1 + dB2 * dB2
            updB = distB > distA
            distP = jnp.maximum(distA, distB)
            jP = jnp.where(updB, jB, jA)
            cur = vmax_ref[rows, cols]
            upd = distP > cur
            vmax_ref[rows, cols] = jnp.maximum(distP, cur)
            vidx_ref[rows, cols] = jnp.where(upd, jP, vidx_ref[rows, cols])

    @pl.when(j2 == NJ // 2 - 1)
    def _finish():
        vm = vmax_ref[...]
        lane = lax.broadcasted_iota(jnp.int32, (BN, BK), 1)
        vi = vidx_ref[...] * BK + lane
        m = jnp.max(vm, axis=1, keepdims=True)
        idx = jnp.min(jnp.where(vm == m, vi, 2**31 - 1), axis=1)
        out_ref[...] = idx.reshape(1, 1, BN)


def _make_argmax_call(rows):
    ni = rows // BN
    return pl.pallas_call(
        _argmax_body,
        grid=(ni, NJ // 2),
        in_specs=[
            pl.BlockSpec((BN, 3), lambda i, j: (i, 0)),
            pl.BlockSpec((3, K), lambda i, j: (0, 0)),
        ],
        out_specs=pl.BlockSpec((1, 1, BN), lambda i, j: (i, 0, 0)),
        out_shape=jax.ShapeDtypeStruct((ni, 1, BN), jnp.int32),
        scratch_shapes=[
            pltpu.VMEM((BN, BK), jnp.float32),
            pltpu.VMEM((BN, BK), jnp.int32),
            pltpu.VMEM((SN, K), jnp.float32),
            pltpu.VMEM((SN, K), jnp.float32),
            pltpu.VMEM((SN, K), jnp.float32),
            pltpu.VMEM((BN, SL), jnp.float32),
            pltpu.VMEM((BN, SL), jnp.float32),
            pltpu.VMEM((BN, SL), jnp.float32),
        ],
        compiler_params=pltpu.CompilerParams(
            dimension_semantics=("parallel", "arbitrary"),
        ),
    )


def _make_sc_gather(rows):
    bpw = rows // _NW
    nch = 4
    ch = bpw // nch

    @functools.partial(
        pl.kernel,
        mesh=_sc_mesh,
        out_type=jax.ShapeDtypeStruct((rows, D), jnp.float32),
        scratch_types=[
            pltpu.VMEM((bpw,), jnp.int32),
            pltpu.VMEM((bpw, D), jnp.float32),
            [pltpu.SemaphoreType.DMA] * nch,
            pltpu.SemaphoreType.DMA,
        ],
    )
    def gather(table_hbm, idx_hbm, out_hbm, idx_v, rows_v, gsems, ssem):
        wid = lax.axis_index("s") * _NC + lax.axis_index("c")
        base = wid * bpw
        pltpu.sync_copy(idx_hbm.at[pl.ds(base, bpw)], idx_v)
        gathers = []
        for c in range(nch):
            gathers.append(
                pltpu.async_copy(
                    table_hbm.at[idx_v.at[pl.ds(c * ch, ch)]],
                    rows_v.at[pl.ds(c * ch, ch)],
                    gsems[c],
                )
            )
        scatters = []
        for c in range(nch):
            gathers[c].wait()
            scatters.append(
                pltpu.async_copy(
                    rows_v.at[pl.ds(c * ch, ch)],
                    out_hbm.at[pl.ds(base + c * ch, ch)],
                    ssem,
                )
            )
        for s in scatters:
            s.wait()

    return gather


_NH = N // H
_argmax_call = _make_argmax_call(_NH)
_sc_gather = _make_sc_gather(_NH)


def kernel(x, anchors, codebook):
    at = anchors.T
    outs = []
    for h in range(H):
        xs = lax.slice(x, (h * _NH, 0), ((h + 1) * _NH, 3))
        idx = _argmax_call(xs, at).reshape(_NH)
        outs.append(_sc_gather(codebook, idx))
    return jnp.concatenate(outs, axis=0)


# speedup vs baseline: 1.0757x; 1.0011x over previous
import functools

import jax
import jax.numpy as jnp
from jax import lax
from jax.experimental import pallas as pl
from jax.experimental.pallas import tpu as pltpu
from jax.experimental.pallas import tpu_sc as plsc

N = 8192
K = 8192
D = 256

BN = 512
BK = 512
NJ = K // BK

SN = 32
SL = 128

H = 8

_NC = 2
_NS = 16
_NW = _NC * _NS

_sc_mesh = plsc.VectorSubcoreMesh(core_axis_name="c", subcore_axis_name="s")


def _argmax_body(x_ref, at_ref, out_ref, vmax_ref, vidx_ref,
                 ab0_ref, ab1_ref, ab2_ref, xb0_ref, xb1_ref, xb2_ref):
    i = pl.program_id(0)
    j2 = pl.program_id(1)
    jA = 2 * j2
    jB = 2 * j2 + 1

    @pl.when((i == 0) & (j2 == 0))
    def _build_ab():
        ab0_ref[...] = jnp.broadcast_to(at_ref[0:1, :], (SN, K))
        ab1_ref[...] = jnp.broadcast_to(at_ref[1:2, :], (SN, K))
        ab2_ref[...] = jnp.broadcast_to(at_ref[2:3, :], (SN, K))

    @pl.when(j2 == 0)
    def _init():
        vmax_ref[...] = jnp.full((BN, BK), -jnp.inf, jnp.float32)
        vidx_ref[...] = jnp.zeros((BN, BK), jnp.int32)
        xb0_ref[...] = jnp.broadcast_to(x_ref[:, 0:1], (BN, SL))
        xb1_ref[...] = jnp.broadcast_to(x_ref[:, 1:2], (BN, SL))
        xb2_ref[...] = jnp.broadcast_to(x_ref[:, 2:3], (BN, SL))

    for li in range(BK // SL):
        cols = pl.ds(li * SL, SL)
        aAcols = pl.ds(jA * BK + li * SL, SL)
        aBcols = pl.ds(jB * BK + li * SL, SL)
        aA0 = ab0_ref[:, aAcols]
        aA1 = ab1_ref[:, aAcols]
        aA2 = ab2_ref[:, aAcols]
        aB0 = ab0_ref[:, aBcols]
        aB1 = ab1_ref[:, aBcols]
        aB2 = ab2_ref[:, aBcols]
        for ni in range(BN // SN):
            rows = pl.ds(ni * SN, SN)
            x0 = xb0_ref[rows, :]
            x1 = xb1_ref[rows, :]
            x2 = xb2_ref[rows, :]
            dA0 = x0 - aA0
            dA1 = x1 - aA1
            dA2 = x2 - aA2
            distA = dA0 * dA0 + dA1 * dA1 + dA2 * dA2
            dB0 = x0 - aB0
            dB1 = x1 - aB1
            dB2 = x2 - aB2
            distB = dB0 * dB0 + dB1 * dB1 + dB2 * dB2
            updB = distB > distA
            distP = jnp.maximum(distA, distB)
            jP = jnp.where(updB, jB, jA)
            cur = vmax_ref[rows, cols]
            upd = distP > cur
            vmax_ref[rows, cols] = jnp.maximum(distP, cur)
            vidx_ref[rows, cols] = jnp.where(upd, jP, vidx_ref[rows, cols])

    @pl.when(j2 == NJ // 2 - 1)
    def _finish():
        vm = vmax_ref[...]
        lane = lax.broadcasted_iota(jnp.int32, (BN, BK), 1)
        vi = vidx_ref[...] * BK + lane
        m = jnp.max(vm, axis=1, keepdims=True)
        idx = jnp.min(jnp.where(vm == m, vi, 2**31 - 1), axis=1)
        out_ref[...] = idx.reshape(1, 1, BN)


def _make_argmax_call(rows):
    ni = rows // BN
    return pl.pallas_call(
        _argmax_body,
        grid=(ni, NJ // 2),
        in_specs=[
            pl.BlockSpec((BN, 3), lambda i, j: (i, 0)),
            pl.BlockSpec((3, K), lambda i, j: (0, 0)),
        ],
        out_specs=pl.BlockSpec((1, 1, BN), lambda i, j: (i, 0, 0)),
        out_shape=jax.ShapeDtypeStruct((ni, 1, BN), jnp.int32),
        scratch_shapes=[
            pltpu.VMEM((BN, BK), jnp.float32),
            pltpu.VMEM((BN, BK), jnp.int32),
            pltpu.VMEM((SN, K), jnp.float32),
            pltpu.VMEM((SN, K), jnp.float32),
            pltpu.VMEM((SN, K), jnp.float32),
            pltpu.VMEM((BN, SL), jnp.float32),
            pltpu.VMEM((BN, SL), jnp.float32),
            pltpu.VMEM((BN, SL), jnp.float32),
        ],
        compiler_params=pltpu.CompilerParams(
            dimension_semantics=("parallel", "arbitrary"),
        ),
    )


def _make_sc_gather(rows):
    bpw = rows // _NW
    nch = 4
    ch = bpw // nch

    @functools.partial(
        pl.kernel,
        mesh=_sc_mesh,
        out_type=jax.ShapeDtypeStruct((rows, D), jnp.float32),
        scratch_types=[
            pltpu.VMEM((bpw,), jnp.int32),
            pltpu.VMEM((bpw, D), jnp.float32),
            [pltpu.SemaphoreType.DMA] * nch,
            pltpu.SemaphoreType.DMA,
        ],
    )
    def gather(table_hbm, idx_hbm, out_hbm, idx_v, rows_v, gsems, ssem):
        wid = lax.axis_index("s") * _NC + lax.axis_index("c")
        base = wid * bpw
        pltpu.sync_copy(idx_hbm.at[pl.ds(base, bpw)], idx_v)
        gathers = []
        for c in range(nch):
            gathers.append(
                pltpu.async_copy(
                    table_hbm.at[idx_v.at[pl.ds(c * ch, ch)]],
                    rows_v.at[pl.ds(c * ch, ch)],
                    gsems[c],
                )
            )
        scatters = []
        for c in range(nch):
            gathers[c].wait()
            scatters.append(
                pltpu.async_copy(
                    rows_v.at[pl.ds(c * ch, ch)],
                    out_hbm.at[pl.ds(base + c * ch, ch)],
                    ssem,
                )
            )
        for s in scatters:
            s.wait()

    return gather


_NH = N // H
_argmax_call = _make_argmax_call(_NH)
_sc_gather = _make_sc_gather(_NH)


def kernel(x, anchors, codebook):
    at = anchors.T
    outs = []
    for h in range(H):
        xs = lax.slice(x, (h * _NH, 0), ((h + 1) * _NH, 3))
        idx = _argmax_call(xs, at).reshape(_NH)
        outs.append(_sc_gather(codebook, idx))
    return jnp.concatenate(outs, axis=0)


# speedup vs baseline: 1.2075x; 1.1225x over previous
import functools

import jax
import jax.numpy as jnp
from jax import lax
from jax.experimental import pallas as pl
from jax.experimental.pallas import tpu as pltpu
from jax.experimental.pallas import tpu_sc as plsc

N = 8192
K = 8192
D = 256

BN = 512
BK = 512
NJ = K // BK

SN = 16
SL = 128
JPB = 4

H = 4

_NC = 2
_NS = 16
_NW = _NC * _NS

_sc_mesh = plsc.VectorSubcoreMesh(core_axis_name="c", subcore_axis_name="s")


def _argmax_body(x_ref, at_ref, out_ref, vmax_ref, vidx_ref,
                 ab0_ref, ab1_ref, ab2_ref, xb0_ref, xb1_ref, xb2_ref):
    i = pl.program_id(0)
    j2 = pl.program_id(1)
    js = [JPB * j2 + q for q in range(JPB)]

    @pl.when((i == 0) & (j2 == 0))
    def _build_ab():
        ab0_ref[...] = jnp.broadcast_to(at_ref[0:1, :], (SN, K))
        ab1_ref[...] = jnp.broadcast_to(at_ref[1:2, :], (SN, K))
        ab2_ref[...] = jnp.broadcast_to(at_ref[2:3, :], (SN, K))

    @pl.when(j2 == 0)
    def _init():
        vmax_ref[...] = jnp.full((BN, BK), -jnp.inf, jnp.float32)
        vidx_ref[...] = jnp.zeros((BN, BK), jnp.int32)
        xb0_ref[...] = jnp.broadcast_to(x_ref[:, 0:1], (BN, SL))
        xb1_ref[...] = jnp.broadcast_to(x_ref[:, 1:2], (BN, SL))
        xb2_ref[...] = jnp.broadcast_to(x_ref[:, 2:3], (BN, SL))

    for li in range(BK // SL):
        cols = pl.ds(li * SL, SL)
        aq = []
        for q in range(JPB):
            acols = pl.ds(js[q] * BK + li * SL, SL)
            aq.append((ab0_ref[:, acols], ab1_ref[:, acols], ab2_ref[:, acols]))
        for ni in range(BN // SN):
            rows = pl.ds(ni * SN, SN)
            x0 = xb0_ref[rows, :]
            x1 = xb1_ref[rows, :]
            x2 = xb2_ref[rows, :]
            vals = []
            for q in range(JPB):
                a0, a1, a2 = aq[q]
                d0 = x0 - a0
                d1 = x1 - a1
                d2 = x2 - a2
                vals.append((d0 * d0 + d1 * d1 + d2 * d2, js[q]))
            while len(vals) > 1:
                nxt = []
                for p in range(0, len(vals), 2):
                    (va, ja), (vb, jb) = vals[p], vals[p + 1]
                    up = vb > va
                    nxt.append((jnp.maximum(va, vb), jnp.where(up, jb, ja)))
                vals = nxt
            distP, jP = vals[0]
            cur = vmax_ref[rows, cols]
            upd = distP > cur
            vmax_ref[rows, cols] = jnp.maximum(distP, cur)
            vidx_ref[rows, cols] = jnp.where(upd, jP, vidx_ref[rows, cols])

    @pl.when(j2 == NJ // JPB - 1)
    def _finish():
        vm = vmax_ref[...]
        lane = lax.broadcasted_iota(jnp.int32, (BN, BK), 1)
        vi = vidx_ref[...] * BK + lane
        m = jnp.max(vm, axis=1, keepdims=True)
        idx = jnp.min(jnp.where(vm == m, vi, 2**31 - 1), axis=1)
        out_ref[...] = idx.reshape(1, 1, BN)


def _make_argmax_call(rows):
    ni = rows // BN
    return pl.pallas_call(
        _argmax_body,
        grid=(ni, NJ // JPB),
        in_specs=[
            pl.BlockSpec((BN, 3), lambda i, j: (i, 0)),
            pl.BlockSpec((3, K), lambda i, j: (0, 0)),
        ],
        out_specs=pl.BlockSpec((1, 1, BN), lambda i, j: (i, 0, 0)),
        out_shape=jax.ShapeDtypeStruct((ni, 1, BN), jnp.int32),
        scratch_shapes=[
            pltpu.VMEM((BN, BK), jnp.float32),
            pltpu.VMEM((BN, BK), jnp.int32),
            pltpu.VMEM((SN, K), jnp.float32),
            pltpu.VMEM((SN, K), jnp.float32),
            pltpu.VMEM((SN, K), jnp.float32),
            pltpu.VMEM((BN, SL), jnp.float32),
            pltpu.VMEM((BN, SL), jnp.float32),
            pltpu.VMEM((BN, SL), jnp.float32),
        ],
        compiler_params=pltpu.CompilerParams(
            dimension_semantics=("parallel", "arbitrary"),
        ),
    )


def _make_sc_gather(rows):
    bpw = rows // _NW
    nch = 4
    ch = bpw // nch

    @functools.partial(
        pl.kernel,
        mesh=_sc_mesh,
        out_type=jax.ShapeDtypeStruct((rows, D), jnp.float32),
        scratch_types=[
            pltpu.VMEM((bpw,), jnp.int32),
            pltpu.VMEM((bpw, D), jnp.float32),
            [pltpu.SemaphoreType.DMA] * nch,
            pltpu.SemaphoreType.DMA,
        ],
    )
    def gather(table_hbm, idx_hbm, out_hbm, idx_v, rows_v, gsems, ssem):
        wid = lax.axis_index("s") * _NC + lax.axis_index("c")
        base = wid * bpw
        pltpu.sync_copy(idx_hbm.at[pl.ds(base, bpw)], idx_v)
        gathers = []
        for c in range(nch):
            gathers.append(
                pltpu.async_copy(
                    table_hbm.at[idx_v.at[pl.ds(c * ch, ch)]],
                    rows_v.at[pl.ds(c * ch, ch)],
                    gsems[c],
                )
            )
        scatters = []
        for c in range(nch):
            gathers[c].wait()
            scatters.append(
                pltpu.async_copy(
                    rows_v.at[pl.ds(c * ch, ch)],
                    out_hbm.at[pl.ds(base + c * ch, ch)],
                    ssem,
                )
            )
        for s in scatters:
            s.wait()

    return gather


_NH = N // H
_argmax_call = _make_argmax_call(_NH)
_sc_gather = _make_sc_gather(_NH)


def kernel(x, anchors, codebook):
    at = anchors.T
    outs = []
    for h in range(H):
        xs = lax.slice(x, (h * _NH, 0), ((h + 1) * _NH, 3))
        idx = _argmax_call(xs, at).reshape(_NH)
        outs.append(_sc_gather(codebook, idx))
    return jnp.concatenate(outs, axis=0)


# speedup vs baseline: 1.2202x; 1.0105x over previous
import functools

import jax
import jax.numpy as jnp
from jax import lax
from jax.experimental import pallas as pl
from jax.experimental.pallas import tpu as pltpu
from jax.experimental.pallas import tpu_sc as plsc

N = 8192
K = 8192
D = 256

BN = 512
BK = 512
NJ = K // BK

SN = 16
SL = 128
JPB = 8

H = 4

_NC = 2
_NS = 16
_NW = _NC * _NS

_sc_mesh = plsc.VectorSubcoreMesh(core_axis_name="c", subcore_axis_name="s")


def _argmax_body(x_ref, at_ref, out_ref, vmax_ref, vidx_ref,
                 ab0_ref, ab1_ref, ab2_ref, xb0_ref, xb1_ref, xb2_ref):
    i = pl.program_id(0)
    j2 = pl.program_id(1)
    js = [JPB * j2 + q for q in range(JPB)]

    @pl.when((i == 0) & (j2 == 0))
    def _build_ab():
        ab0_ref[...] = jnp.broadcast_to(at_ref[0:1, :], (SN, K))
        ab1_ref[...] = jnp.broadcast_to(at_ref[1:2, :], (SN, K))
        ab2_ref[...] = jnp.broadcast_to(at_ref[2:3, :], (SN, K))

    @pl.when(j2 == 0)
    def _init():
        vmax_ref[...] = jnp.full((BN, BK), -jnp.inf, jnp.float32)
        vidx_ref[...] = jnp.zeros((BN, BK), jnp.int32)
        xb0_ref[...] = jnp.broadcast_to(x_ref[:, 0:1], (BN, SL))
        xb1_ref[...] = jnp.broadcast_to(x_ref[:, 1:2], (BN, SL))
        xb2_ref[...] = jnp.broadcast_to(x_ref[:, 2:3], (BN, SL))

    for li in range(BK // SL):
        cols = pl.ds(li * SL, SL)
        aq = []
        for q in range(JPB):
            acols = pl.ds(js[q] * BK + li * SL, SL)
            aq.append((ab0_ref[:, acols], ab1_ref[:, acols], ab2_ref[:, acols]))
        for ni in range(BN // SN):
            rows = pl.ds(ni * SN, SN)
            x0 = xb0_ref[rows, :]
            x1 = xb1_ref[rows, :]
            x2 = xb2_ref[rows, :]
            vals = []
            for q in range(JPB):
                a0, a1, a2 = aq[q]
                d0 = x0 - a0
                d1 = x1 - a1
                d2 = x2 - a2
                vals.append((d0 * d0 + d1 * d1 + d2 * d2, js[q]))
            while len(vals) > 1:
                nxt = []
                for p in range(0, len(vals), 2):
                    (va, ja), (vb, jb) = vals[p], vals[p + 1]
                    up = vb > va
                    nxt.append((jnp.maximum(va, vb), jnp.where(up, jb, ja)))
                vals = nxt
            distP, jP = vals[0]
            cur = vmax_ref[rows, cols]
            upd = distP > cur
            vmax_ref[rows, cols] = jnp.maximum(distP, cur)
            vidx_ref[rows, cols] = jnp.where(upd, jP, vidx_ref[rows, cols])

    @pl.when(j2 == NJ // JPB - 1)
    def _finish():
        vm = vmax_ref[...]
        lane = lax.broadcasted_iota(jnp.int32, (BN, BK), 1)
        vi = vidx_ref[...] * BK + lane
        m = jnp.max(vm, axis=1, keepdims=True)
        idx = jnp.min(jnp.where(vm == m, vi, 2**31 - 1), axis=1)
        out_ref[...] = idx.reshape(1, 1, BN)


def _make_argmax_call(rows):
    ni = rows // BN
    return pl.pallas_call(
        _argmax_body,
        grid=(ni, NJ // JPB),
        in_specs=[
            pl.BlockSpec((BN, 3), lambda i, j: (i, 0)),
            pl.BlockSpec((3, K), lambda i, j: (0, 0)),
        ],
        out_specs=pl.BlockSpec((1, 1, BN), lambda i, j: (i, 0, 0)),
        out_shape=jax.ShapeDtypeStruct((ni, 1, BN), jnp.int32),
        scratch_shapes=[
            pltpu.VMEM((BN, BK), jnp.float32),
            pltpu.VMEM((BN, BK), jnp.int32),
            pltpu.VMEM((SN, K), jnp.float32),
            pltpu.VMEM((SN, K), jnp.float32),
            pltpu.VMEM((SN, K), jnp.float32),
            pltpu.VMEM((BN, SL), jnp.float32),
            pltpu.VMEM((BN, SL), jnp.float32),
            pltpu.VMEM((BN, SL), jnp.float32),
        ],
        compiler_params=pltpu.CompilerParams(
            dimension_semantics=("parallel", "arbitrary"),
        ),
    )


def _make_sc_gather(rows):
    bpw = rows // _NW
    nch = 4
    ch = bpw // nch

    @functools.partial(
        pl.kernel,
        mesh=_sc_mesh,
        out_type=jax.ShapeDtypeStruct((rows, D), jnp.float32),
        scratch_types=[
            pltpu.VMEM((bpw,), jnp.int32),
            pltpu.VMEM((bpw, D), jnp.float32),
            [pltpu.SemaphoreType.DMA] * nch,
            pltpu.SemaphoreType.DMA,
        ],
    )
    def gather(table_hbm, idx_hbm, out_hbm, idx_v, rows_v, gsems, ssem):
        wid = lax.axis_index("s") * _NC + lax.axis_index("c")
        base = wid * bpw
        pltpu.sync_copy(idx_hbm.at[pl.ds(base, bpw)], idx_v)
        gathers = []
        for c in range(nch):
            gathers.append(
                pltpu.async_copy(
                    table_hbm.at[idx_v.at[pl.ds(c * ch, ch)]],
                    rows_v.at[pl.ds(c * ch, ch)],
                    gsems[c],
                )
            )
        scatters = []
        for c in range(nch):
            gathers[c].wait()
            scatters.append(
                pltpu.async_copy(
                    rows_v.at[pl.ds(c * ch, ch)],
                    out_hbm.at[pl.ds(base + c * ch, ch)],
                    ssem,
                )
            )
        for s in scatters:
            s.wait()

    return gather


_NH = N // H
_argmax_call = _make_argmax_call(_NH)
_sc_gather = _make_sc_gather(_NH)


def kernel(x, anchors, codebook):
    at = anchors.T
    outs = []
    for h in range(H):
        xs = lax.slice(x, (h * _NH, 0), ((h + 1) * _NH, 3))
        idx = _argmax_call(xs, at).reshape(_NH)
        outs.append(_sc_gather(codebook, idx))
    return jnp.concatenate(outs, axis=0)


# speedup vs baseline: 1.2227x; 1.0020x over previous
import functools

import jax
import jax.numpy as jnp
from jax import lax
from jax.experimental import pallas as pl
from jax.experimental.pallas import tpu as pltpu
from jax.experimental.pallas import tpu_sc as plsc

N = 8192
K = 8192
D = 256

BN = 512
BK = 512
NJ = K // BK

SN = 16
SL = 128
JPB = 8

H = 4

_NC = 2
_NS = 16
_NW = _NC * _NS

_sc_mesh = plsc.VectorSubcoreMesh(core_axis_name="c", subcore_axis_name="s")


def _argmax_body(x_ref, at_ref, out_ref, vmax_ref, vidx_ref,
                 ab0_ref, ab1_ref, ab2_ref, xb0_ref, xb1_ref, xb2_ref):
    i = pl.program_id(0)
    j2 = pl.program_id(1)
    js = [JPB * j2 + q for q in range(JPB)]

    @pl.when((i == 0) & (j2 == 0))
    def _build_ab():
        ab0_ref[...] = jnp.broadcast_to(at_ref[0:1, :], (SN, K))
        ab1_ref[...] = jnp.broadcast_to(at_ref[1:2, :], (SN, K))
        ab2_ref[...] = jnp.broadcast_to(at_ref[2:3, :], (SN, K))

    @pl.when(j2 == 0)
    def _init():
        vmax_ref[...] = jnp.full((BN, BK), -jnp.inf, jnp.float32)
        vidx_ref[...] = jnp.zeros((BN, BK), jnp.int32)
        xb0_ref[...] = jnp.broadcast_to(x_ref[:, 0:1], (BN, SL))
        xb1_ref[...] = jnp.broadcast_to(x_ref[:, 1:2], (BN, SL))
        xb2_ref[...] = jnp.broadcast_to(x_ref[:, 2:3], (BN, SL))

    for li in range(BK // SL):
        cols = pl.ds(li * SL, SL)
        aq = []
        for q in range(JPB):
            acols = pl.ds(js[q] * BK + li * SL, SL)
            aq.append((ab0_ref[:, acols], ab1_ref[:, acols], ab2_ref[:, acols]))
        for ni in range(BN // SN):
            rows = pl.ds(ni * SN, SN)
            x0 = xb0_ref[rows, :]
            x1 = xb1_ref[rows, :]
            x2 = xb2_ref[rows, :]
            vals = []
            for q in range(JPB):
                a0, a1, a2 = aq[q]
                d0 = x0 - a0
                d1 = x1 - a1
                d2 = x2 - a2
                vals.append((d0 * d0 + d1 * d1 + d2 * d2, js[q]))
            while len(vals) > 1:
                nxt = []
                for p in range(0, len(vals), 2):
                    (va, ja), (vb, jb) = vals[p], vals[p + 1]
                    up = vb > va
                    nxt.append((jnp.maximum(va, vb), jnp.where(up, jb, ja)))
                vals = nxt
            distP, jP = vals[0]
            cur = vmax_ref[rows, cols]
            upd = distP > cur
            vmax_ref[rows, cols] = jnp.maximum(distP, cur)
            vidx_ref[rows, cols] = jnp.where(upd, jP, vidx_ref[rows, cols])

    @pl.when(j2 == NJ // JPB - 1)
    def _finish():
        vm = vmax_ref[...]
        lane = lax.broadcasted_iota(jnp.int32, (BN, BK), 1)
        vi = vidx_ref[...] * BK + lane
        m = jnp.max(vm, axis=1, keepdims=True)
        idx = jnp.min(jnp.where(vm == m, vi, 2**31 - 1), axis=1)
        out_ref[...] = idx.reshape(1, 1, BN)


def _make_argmax_call(rows):
    ni = rows // BN
    return pl.pallas_call(
        _argmax_body,
        grid=(ni, NJ // JPB),
        in_specs=[
            pl.BlockSpec((BN, 3), lambda i, j: (i, 0)),
            pl.BlockSpec((3, K), lambda i, j: (0, 0)),
        ],
        out_specs=pl.BlockSpec((1, 1, BN), lambda i, j: (i, 0, 0)),
        out_shape=jax.ShapeDtypeStruct((ni, 1, BN), jnp.int32),
        scratch_shapes=[
            pltpu.VMEM((BN, BK), jnp.float32),
            pltpu.VMEM((BN, BK), jnp.int32),
            pltpu.VMEM((SN, K), jnp.float32),
            pltpu.VMEM((SN, K), jnp.float32),
            pltpu.VMEM((SN, K), jnp.float32),
            pltpu.VMEM((BN, SL), jnp.float32),
            pltpu.VMEM((BN, SL), jnp.float32),
            pltpu.VMEM((BN, SL), jnp.float32),
        ],
        compiler_params=pltpu.CompilerParams(
            dimension_semantics=("parallel", "arbitrary"),
        ),
    )


def _make_sc_gather(rows):
    bpw = rows // _NW
    nch = 1
    ch = bpw // nch

    @functools.partial(
        pl.kernel,
        mesh=_sc_mesh,
        out_type=jax.ShapeDtypeStruct((rows, D), jnp.float32),
        scratch_types=[
            pltpu.VMEM((bpw,), jnp.int32),
            pltpu.VMEM((bpw, D), jnp.float32),
            [pltpu.SemaphoreType.DMA] * nch,
            pltpu.SemaphoreType.DMA,
        ],
    )
    def gather(table_hbm, idx_hbm, out_hbm, idx_v, rows_v, gsems, ssem):
        wid = lax.axis_index("s") * _NC + lax.axis_index("c")
        base = wid * bpw
        pltpu.sync_copy(idx_hbm.at[pl.ds(base, bpw)], idx_v)
        gathers = []
        for c in range(nch):
            gathers.append(
                pltpu.async_copy(
                    table_hbm.at[idx_v.at[pl.ds(c * ch, ch)]],
                    rows_v.at[pl.ds(c * ch, ch)],
                    gsems[c],
                )
            )
        scatters = []
        for c in range(nch):
            gathers[c].wait()
            scatters.append(
                pltpu.async_copy(
                    rows_v.at[pl.ds(c * ch, ch)],
                    out_hbm.at[pl.ds(base + c * ch, ch)],
                    ssem,
                )
            )
        for s in scatters:
            s.wait()

    return gather


_NH = N // H
_argmax_call = _make_argmax_call(_NH)
_sc_gather = _make_sc_gather(_NH)


def kernel(x, anchors, codebook):
    at = anchors.T
    outs = []
    for h in range(H):
        xs = lax.slice(x, (h * _NH, 0), ((h + 1) * _NH, 3))
        idx = _argmax_call(xs, at).reshape(_NH)
        outs.append(_sc_gather(codebook, idx))
    return jnp.concatenate(outs, axis=0)
